# Initial kernel scaffold; baseline (speedup 1.0000x reference)
import jax
import jax.numpy as jnp
from jax.experimental import pallas as pl


def kernel(x, edge_index, params):
    raise NotImplementedError("write your pallas kernel here")


# speedup vs baseline: 2.6585x; 2.6585x over previous
import functools

import jax
import jax.numpy as jnp
from jax import lax
from jax.experimental import pallas as pl
from jax.experimental.pallas import tpu as pltpu
from jax.experimental.pallas import tpu_sc as plsc

N = 10000
NP = 10240
E = 160000
H = 8
NC = 2
NS = 16
LANES = 16
PK = 16384

f32 = jnp.float32
i32 = jnp.int32


def _mm(z, w):
    M, K = z.shape
    _, Nn = w.shape
    bm, bk = 256, 256
    bn = 256 if Nn % 256 == 0 else 128
    gm, gn, gk = M // bm, Nn // bn, K // bk

    def body(zb, wb, ob):
        @pl.when(pl.program_id(2) == 0)
        def _():
            ob[...] = jnp.zeros_like(ob)
        ob[...] += jnp.dot(zb[...], wb[...], preferred_element_type=f32)

    return pl.pallas_call(
        body,
        grid=(gm, gn, gk),
        in_specs=[pl.BlockSpec((bm, bk), lambda i, j, k: (i, k)),
                  pl.BlockSpec((bk, bn), lambda i, j, k: (k, j))],
        out_specs=pl.BlockSpec((bm, bn), lambda i, j, k: (i, j)),
        out_shape=jax.ShapeDtypeStruct((M, Nn), f32),
        compiler_params=pltpu.CompilerParams(
            dimension_semantics=("parallel", "parallel", "arbitrary")),
    )(z, w)


def _att(h0, h1, a_src, a_dst, Hh, C):
    M, D2 = h0.shape
    Hh2 = Hh // 2
    bm = 256

    def body(h0b, h1b, asb, adb, ob):
        cols = []
        for vec, base in ((asb, 0), (adb, 8)):
            del base
            for k in range(Hh):
                hb = h0b if k < Hh2 else h1b
                kl = k % Hh2
                hs = hb[:, kl * C:(kl + 1) * C]
                cols.append(jnp.sum(hs * vec[0, k * C:(k + 1) * C][None, :],
                                    axis=1, keepdims=True))
            for _ in range(Hh, 8):
                cols.append(jnp.zeros((bm, 1), f32))
        ob[...] = jnp.concatenate(cols, axis=1)

    return pl.pallas_call(
        body,
        grid=(M // bm,),
        in_specs=[pl.BlockSpec((bm, D2), lambda i: (i, 0)),
                  pl.BlockSpec((bm, D2), lambda i: (i, 0)),
                  pl.BlockSpec((1, Hh * C), lambda i: (0, 0)),
                  pl.BlockSpec((1, Hh * C), lambda i: (0, 0))],
        out_specs=pl.BlockSpec((bm, 16), lambda i: (i, 0)),
        out_shape=jax.ShapeDtypeStruct((M, 16), f32),
    )(h0, h1, a_src.reshape(1, -1), a_dst.reshape(1, -1))


def _norm(acc0, acc1, accw0, accw1, b, g, lb, Hh, C):
    M, D2 = acc0.shape
    D = 2 * D2
    Hh2 = Hh // 2
    bm = 256

    def body(a0, a1, w0, w1, bb, gb, lbb, ob):
        s = w0[...] + w1[...]
        parts = []
        for k in range(Hh):
            ab = a0 if k < Hh2 else a1
            kl = k % Hh2
            parts.append(ab[:, kl * C:(kl + 1) * C]
                         / (s[:, k:k + 1] + 1e-16))
        y = jnp.concatenate(parts, axis=1) + bb[0][None, :]
        y = jnp.maximum(y, 0.0)
        mu = jnp.mean(y, axis=1, keepdims=True)
        var = jnp.mean((y - mu) * (y - mu), axis=1, keepdims=True)
        ob[...] = (y - mu) * lax.rsqrt(var + 1e-5) * gb[0][None, :] + lbb[0][None, :]

    return pl.pallas_call(
        body,
        grid=(M // bm,),
        in_specs=[pl.BlockSpec((bm, D2), lambda i: (i, 0)),
                  pl.BlockSpec((bm, D2), lambda i: (i, 0)),
                  pl.BlockSpec((bm, 16), lambda i: (i, 0)),
                  pl.BlockSpec((bm, 16), lambda i: (i, 0)),
                  pl.BlockSpec((1, D), lambda i: (0, 0)),
                  pl.BlockSpec((1, D), lambda i: (0, 0)),
                  pl.BlockSpec((1, D), lambda i: (0, 0))],
        out_specs=pl.BlockSpec((bm, D), lambda i: (i, 0)),
        out_shape=jax.ShapeDtypeStruct((M, D), f32),
    )(acc0, acc1, accw0, accw1,
      b.reshape(1, -1), g.reshape(1, -1), lb.reshape(1, -1))


def _final(acc0, acc1, accw0, accw1, b, Hh, C):
    M, D2 = acc0.shape
    D = 2 * D2
    Hh2 = Hh // 2
    bm = 256

    def body(a0, a1, w0, w1, bb, ob):
        s = w0[...] + w1[...]
        parts = []
        for k in range(Hh):
            ab = a0 if k < Hh2 else a1
            kl = k % Hh2
            parts.append(ab[:, kl * C:(kl + 1) * C]
                         / (s[:, k:k + 1] + 1e-16))
        ob[...] = jnp.concatenate(parts, axis=1) + bb[0][None, :]

    return pl.pallas_call(
        body,
        grid=(M // bm,),
        in_specs=[pl.BlockSpec((bm, D2), lambda i: (i, 0)),
                  pl.BlockSpec((bm, D2), lambda i: (i, 0)),
                  pl.BlockSpec((bm, 16), lambda i: (i, 0)),
                  pl.BlockSpec((bm, 16), lambda i: (i, 0)),
                  pl.BlockSpec((1, D), lambda i: (0, 0))],
        out_specs=pl.BlockSpec((bm, D), lambda i: (i, 0)),
        out_shape=jax.ShapeDtypeStruct((M, D), f32),
    )(acc0, acc1, accw0, accw1, b.reshape(1, -1))


def _sc_edge(pedge, att, h0, h1, zrows, zw, Hh, C, RB, NB):
    D2 = Hh // 2 * C
    Hh2 = Hh // 2
    EPT = E // NS
    NGR = EPT // LANES
    CV = C // LANES
    RPT = RB // NS

    mesh = plsc.VectorSubcoreMesh(core_axis_name="c", subcore_axis_name="s",
                                  num_cores=NC, num_subcores=NS)

    @functools.partial(
        pl.kernel,
        out_type=[jax.ShapeDtypeStruct((NP, D2), f32),
                  jax.ShapeDtypeStruct((NP, D2), f32),
                  jax.ShapeDtypeStruct((NP, 16), f32),
                  jax.ShapeDtypeStruct((NP, 16), f32)],
        mesh=mesh,
        compiler_params=pltpu.CompilerParams(needs_layout_passes=False,
                                             use_tc_tiling_on_sc=False),
        scratch_types=[
            pltpu.VMEM((EPT,), i32),
            pltpu.VMEM((EPT + 16,), i32),
            pltpu.VMEM((16, 16), f32),
            pltpu.VMEM((16, 16), f32),
            pltpu.VMEM((16, 16), f32),
            pltpu.VMEM((16, D2), f32),
            pltpu.VMEM((16,), i32),
            pltpu.VMEM_SHARED((RB, D2), f32),
            pltpu.VMEM_SHARED((RB, 16), f32),
        ],
    )
    def body(pedge_hbm, att_hbm, h0_hbm, h1_hbm, z_hbm, zw_hbm,
             acc0_hbm, acc1_hbm, accw0_hbm, accw1_hbm,
             pedge_v, cpk, asbuf, adbuf, wbuf, hbuf, idxw, acc_sh, accw_sh):
        cid = lax.axis_index("c")
        sid = lax.axis_index("s")

        def bcast(v):
            return lax.broadcast(v, (LANES,))

        pltpu.sync_copy(pedge_hbm.at[pl.ds(sid * EPT, EPT)], pedge_v)
        for r in range(16):
            wbuf[r, :] = jnp.zeros((LANES,), f32)

        def blk(bi, _):
            lo = bi * RB
            hi = lo + RB

            pltpu.sync_copy(z_hbm.at[pl.ds(sid * RPT, RPT)],
                            acc_sh.at[pl.ds(sid * RPT, RPT)])
            pltpu.sync_copy(zw_hbm.at[pl.ds(sid * RPT, RPT)],
                            accw_sh.at[pl.ds(sid * RPT, RPT)])
            plsc.subcore_barrier()

            def scan(g, ptr):
                sl = pl.ds(g * LANES, LANES)
                p16 = pedge_v[sl]
                d16 = p16 & jnp.full((LANES,), PK - 1, i32)
                inb = (d16 >= bcast(lo)) & (d16 < bcast(hi))
                cs = plsc.cumsum(inb.astype(i32))
                pos = jnp.where(inb, bcast(ptr) + cs - 1,
                                jnp.full((LANES,), EPT + 8, i32))
                plsc.store_scatter(cpk, [pos], p16 - bcast(lo))
                cnt = plsc.all_reduce_population_count(inb)
                if cnt.ndim:
                    cnt = cnt[0]
                return ptr + cnt
            ecnt = lax.fori_loop(0, NGR, scan, jnp.int32(0))

            ngrp = (ecnt + LANES - 1) // LANES

            def proc(g, _):
                lane = lax.iota(i32, LANES)
                sl = pl.ds(g * LANES, LANES)
                valid = (bcast(g * LANES) + lane) < bcast(ecnt)
                p16 = jnp.where(valid, cpk[sl], 0)
                s16 = jnp.clip(
                    lax.shift_right_logical(p16, jnp.full((LANES,), 14, i32)),
                    0, N - 1)
                dl16 = jnp.clip(p16 & jnp.full((LANES,), PK - 1, i32),
                                0, RB - 1)
                dg16 = jnp.clip(dl16 + bcast(lo), 0, NP - 1)
                pltpu.sync_copy(att_hbm.at[s16], asbuf)
                pltpu.sync_copy(att_hbm.at[dg16], adbuf)

                @pl.when(cid == 0)
                def _():
                    pltpu.sync_copy(h0_hbm.at[s16], hbuf)

                @pl.when(cid == 1)
                def _():
                    pltpu.sync_copy(h1_hbm.at[s16], hbuf)

                for k in range(Hh2):
                    kg = bcast(cid * Hh2 + k)
                    vs = plsc.load_gather(asbuf, [lane, kg])
                    vd = plsc.load_gather(adbuf, [lane, kg + 8])
                    ev = vs + vd
                    ev = jnp.where(ev >= 0.0, ev, 0.2 * ev)
                    w = jnp.where(valid, jnp.exp(ev), 0.0)
                    plsc.store_scatter(wbuf, [lane, kg], w)

                def colscale(j, _):
                    kg = bcast(cid * Hh2 + j // CV)
                    off = j * LANES
                    for e in range(16):
                        wsv = plsc.load_gather(
                            wbuf, [jnp.full((LANES,), e, i32), kg])
                        hbuf[e, pl.ds(off, LANES)] = (
                            hbuf[e, pl.ds(off, LANES)] * wsv)
                    return 0
                lax.fori_loop(0, D2 // LANES, colscale, 0)

                idxw[...] = dl16
                pltpu.sync_copy(hbuf, acc_sh.at[idxw], add=True)
                pltpu.sync_copy(wbuf, accw_sh.at[idxw], add=True)
                return 0
            lax.fori_loop(0, ngrp, proc, 0)

            plsc.subcore_barrier()

            rows_sh = pl.ds(sid * RPT, RPT)
            rows_g = pl.ds(lo + sid * RPT, RPT)

            @pl.when(cid == 0)
            def _():
                pltpu.sync_copy(acc_sh.at[rows_sh], acc0_hbm.at[rows_g])
                pltpu.sync_copy(accw_sh.at[rows_sh], accw0_hbm.at[rows_g])

            @pl.when(cid == 1)
            def _():
                pltpu.sync_copy(acc_sh.at[rows_sh], acc1_hbm.at[rows_g])
                pltpu.sync_copy(accw_sh.at[rows_sh], accw1_hbm.at[rows_g])
            plsc.subcore_barrier()
            return 0

        lax.fori_loop(0, NB, blk, 0)

    return body(pedge, att, h0, h1, zrows, zw)


TRUNK_OUT = [448, 384, 256]
SC_CFG = {448: (320, 32), 384: (320, 32), 256: (640, 16), 64: (5120, 2)}


def _layer(h, pedge, W, a_src, a_dst, Hh, C):
    D2 = Hh // 2 * C
    h0 = _mm(h, W[:, :D2])
    h1 = _mm(h, W[:, D2:])
    att = _att(h0, h1, a_src, a_dst, Hh, C)
    RB, NB = SC_CFG[C]
    zrows = jnp.zeros((RB, D2), f32)
    zw = jnp.zeros((RB, 16), f32)
    return _sc_edge(pedge, att, h0, h1, zrows, zw, Hh, C, RB, NB)


def kernel(x, edge_index, params):
    xp = jnp.zeros((NP, x.shape[1]), f32).at[:N].set(x)
    pedge = edge_index[0] * PK + edge_index[1]

    h = xp
    for i in range(3):
        p = params["trunk"][i]
        C = TRUNK_OUT[i]
        acc0, acc1, accw0, accw1 = _layer(
            h, pedge, p["W"], p["att_src"].reshape(-1),
            p["att_dst"].reshape(-1), H, C)
        h = _norm(acc0, acc1, accw0, accw1,
                  p["b"], p["ln_g"], p["ln_b"], H, C)

    hp = params["heads"]
    Wf = jnp.concatenate([q["W"] for q in hp], axis=1)
    asf = jnp.concatenate([q["att_src"].reshape(-1) for q in hp], axis=0)
    adf = jnp.concatenate([q["att_dst"].reshape(-1) for q in hp], axis=0)
    bf = jnp.concatenate([q["b"] for q in hp], axis=0)
    acc0, acc1, accw0, accw1 = _layer(h, pedge, Wf, asf, adf, 4, 64)
    out = _final(acc0, acc1, accw0, accw1, bf, 4, 64)
    return out[:N]


# speedup vs baseline: 3.8181x; 1.4362x over previous
import functools

import jax
import jax.numpy as jnp
from jax import lax
from jax.experimental import pallas as pl
from jax.experimental.pallas import tpu as pltpu
from jax.experimental.pallas import tpu_sc as plsc

N = 10000
NP = 10240
E = 160000
H = 8
NC = 2
NS = 16
LANES = 16
PK = 16384

f32 = jnp.float32
i32 = jnp.int32


def _mm(z, w):
    M, K = z.shape
    _, Nn = w.shape
    bm, bk = 256, 256
    bn = 256 if Nn % 256 == 0 else 128
    gm, gn, gk = M // bm, Nn // bn, K // bk

    def body(zb, wb, ob):
        @pl.when(pl.program_id(2) == 0)
        def _():
            ob[...] = jnp.zeros_like(ob)
        ob[...] += jnp.dot(zb[...], wb[...], preferred_element_type=f32)

    return pl.pallas_call(
        body,
        grid=(gm, gn, gk),
        in_specs=[pl.BlockSpec((bm, bk), lambda i, j, k: (i, k)),
                  pl.BlockSpec((bk, bn), lambda i, j, k: (k, j))],
        out_specs=pl.BlockSpec((bm, bn), lambda i, j, k: (i, j)),
        out_shape=jax.ShapeDtypeStruct((M, Nn), f32),
        compiler_params=pltpu.CompilerParams(
            dimension_semantics=("parallel", "parallel", "arbitrary")),
    )(z, w)


def _att(h0, h1, a_src, a_dst, Hh, C):
    M, D2 = h0.shape
    Hh2 = Hh // 2
    bm = 256

    def body(h0b, h1b, asb, adb, ob):
        cols = []
        for vec, base in ((asb, 0), (adb, 8)):
            del base
            for k in range(Hh):
                hb = h0b if k < Hh2 else h1b
                kl = k % Hh2
                hs = hb[:, kl * C:(kl + 1) * C]
                cols.append(jnp.sum(hs * vec[0, k * C:(k + 1) * C][None, :],
                                    axis=1, keepdims=True))
            for _ in range(Hh, 8):
                cols.append(jnp.zeros((bm, 1), f32))
        ob[...] = jnp.concatenate(cols, axis=1)

    return pl.pallas_call(
        body,
        grid=(M // bm,),
        in_specs=[pl.BlockSpec((bm, D2), lambda i: (i, 0)),
                  pl.BlockSpec((bm, D2), lambda i: (i, 0)),
                  pl.BlockSpec((1, Hh * C), lambda i: (0, 0)),
                  pl.BlockSpec((1, Hh * C), lambda i: (0, 0))],
        out_specs=pl.BlockSpec((bm, 16), lambda i: (i, 0)),
        out_shape=jax.ShapeDtypeStruct((M, 16), f32),
    )(h0, h1, a_src.reshape(1, -1), a_dst.reshape(1, -1))


def _norm(acc0, acc1, accw0, accw1, b, g, lb, Hh, C):
    M, D2 = acc0.shape
    D = 2 * D2
    Hh2 = Hh // 2
    bm = 256

    def body(a0, a1, w0, w1, bb, gb, lbb, ob):
        s = w0[...] + w1[...]
        parts = []
        for k in range(Hh):
            ab = a0 if k < Hh2 else a1
            kl = k % Hh2
            parts.append(ab[:, kl * C:(kl + 1) * C]
                         / (s[:, k:k + 1] + 1e-16))
        y = jnp.concatenate(parts, axis=1) + bb[0][None, :]
        y = jnp.maximum(y, 0.0)
        mu = jnp.mean(y, axis=1, keepdims=True)
        var = jnp.mean((y - mu) * (y - mu), axis=1, keepdims=True)
        ob[...] = (y - mu) * lax.rsqrt(var + 1e-5) * gb[0][None, :] + lbb[0][None, :]

    return pl.pallas_call(
        body,
        grid=(M // bm,),
        in_specs=[pl.BlockSpec((bm, D2), lambda i: (i, 0)),
                  pl.BlockSpec((bm, D2), lambda i: (i, 0)),
                  pl.BlockSpec((bm, 16), lambda i: (i, 0)),
                  pl.BlockSpec((bm, 16), lambda i: (i, 0)),
                  pl.BlockSpec((1, D), lambda i: (0, 0)),
                  pl.BlockSpec((1, D), lambda i: (0, 0)),
                  pl.BlockSpec((1, D), lambda i: (0, 0))],
        out_specs=pl.BlockSpec((bm, D), lambda i: (i, 0)),
        out_shape=jax.ShapeDtypeStruct((M, D), f32),
    )(acc0, acc1, accw0, accw1,
      b.reshape(1, -1), g.reshape(1, -1), lb.reshape(1, -1))


def _final(acc0, acc1, accw0, accw1, b, Hh, C):
    M, D2 = acc0.shape
    D = 2 * D2
    Hh2 = Hh // 2
    bm = 256

    def body(a0, a1, w0, w1, bb, ob):
        s = w0[...] + w1[...]
        parts = []
        for k in range(Hh):
            ab = a0 if k < Hh2 else a1
            kl = k % Hh2
            parts.append(ab[:, kl * C:(kl + 1) * C]
                         / (s[:, k:k + 1] + 1e-16))
        ob[...] = jnp.concatenate(parts, axis=1) + bb[0][None, :]

    return pl.pallas_call(
        body,
        grid=(M // bm,),
        in_specs=[pl.BlockSpec((bm, D2), lambda i: (i, 0)),
                  pl.BlockSpec((bm, D2), lambda i: (i, 0)),
                  pl.BlockSpec((bm, 16), lambda i: (i, 0)),
                  pl.BlockSpec((bm, 16), lambda i: (i, 0)),
                  pl.BlockSpec((1, D), lambda i: (0, 0))],
        out_specs=pl.BlockSpec((bm, D), lambda i: (i, 0)),
        out_shape=jax.ShapeDtypeStruct((M, D), f32),
    )(acc0, acc1, accw0, accw1, b.reshape(1, -1))


def _sc_edge(pedge, att, h0, h1, zrows, zw, Hh, C, RB, NB):
    D2 = Hh // 2 * C
    Hh2 = Hh // 2
    EPT = E // NS
    NGR = EPT // LANES
    CV = C // LANES
    RPT = RB // NS

    mesh = plsc.VectorSubcoreMesh(core_axis_name="c", subcore_axis_name="s",
                                  num_cores=NC, num_subcores=NS)

    @functools.partial(
        pl.kernel,
        out_type=[jax.ShapeDtypeStruct((NP, D2), f32),
                  jax.ShapeDtypeStruct((NP, D2), f32),
                  jax.ShapeDtypeStruct((NP, 16), f32),
                  jax.ShapeDtypeStruct((NP, 16), f32)],
        mesh=mesh,
        compiler_params=pltpu.CompilerParams(needs_layout_passes=False,
                                             use_tc_tiling_on_sc=False),
        scratch_types=[
            pltpu.VMEM((EPT,), i32),
            pltpu.VMEM((EPT + 16,), i32),
            pltpu.VMEM((16, 16), f32),
            pltpu.VMEM((16, 16), f32),
            pltpu.VMEM((16, 16), f32),
            pltpu.VMEM((16, D2), f32),
            pltpu.VMEM((16,), i32),
            pltpu.VMEM_SHARED((RB, D2), f32),
            pltpu.VMEM_SHARED((RB, 16), f32),
        ],
    )
    def body(pedge_hbm, att_hbm, h0_hbm, h1_hbm, z_hbm, zw_hbm,
             acc0_hbm, acc1_hbm, accw0_hbm, accw1_hbm,
             pedge_v, cpk, asbuf, adbuf, wbuf, hbuf, idxw, acc_sh, accw_sh):
        cid = lax.axis_index("c")
        sid = lax.axis_index("s")

        def bcast(v):
            return lax.broadcast(v, (LANES,))

        pltpu.sync_copy(pedge_hbm.at[pl.ds(sid * EPT, EPT)], pedge_v)
        for r in range(16):
            wbuf[r, :] = jnp.zeros((LANES,), f32)

        def blk(bi, _):
            lo = bi * RB
            hi = lo + RB

            pltpu.sync_copy(z_hbm.at[pl.ds(sid * RPT, RPT)],
                            acc_sh.at[pl.ds(sid * RPT, RPT)])
            pltpu.sync_copy(zw_hbm.at[pl.ds(sid * RPT, RPT)],
                            accw_sh.at[pl.ds(sid * RPT, RPT)])
            plsc.subcore_barrier()

            def scan(g, ptr):
                sl = pl.ds(g * LANES, LANES)
                p16 = pedge_v[sl]
                d16 = p16 & jnp.full((LANES,), PK - 1, i32)
                inb = (d16 >= bcast(lo)) & (d16 < bcast(hi))
                cs = plsc.cumsum(inb.astype(i32))
                pos = jnp.where(inb, bcast(ptr) + cs - 1,
                                jnp.full((LANES,), EPT + 8, i32))
                plsc.store_scatter(cpk, [pos], p16 - bcast(lo))
                cnt = plsc.all_reduce_population_count(inb)
                if cnt.ndim:
                    cnt = cnt[0]
                return ptr + cnt
            ecnt = lax.fori_loop(0, NGR, scan, jnp.int32(0))

            ngrp = (ecnt + LANES - 1) // LANES

            def proc(g, _):
                lane = lax.iota(i32, LANES)
                sl = pl.ds(g * LANES, LANES)
                valid = (bcast(g * LANES) + lane) < bcast(ecnt)
                p16 = jnp.where(valid, cpk[sl], 0)
                s16 = jnp.clip(
                    lax.shift_right_logical(p16, jnp.full((LANES,), 14, i32)),
                    0, N - 1)
                dl16 = jnp.clip(p16 & jnp.full((LANES,), PK - 1, i32),
                                0, RB - 1)
                dg16 = jnp.clip(dl16 + bcast(lo), 0, NP - 1)
                pltpu.sync_copy(att_hbm.at[s16], asbuf)
                pltpu.sync_copy(att_hbm.at[dg16], adbuf)

                @pl.when(cid == 0)
                def _():
                    pltpu.sync_copy(h0_hbm.at[s16], hbuf)

                @pl.when(cid == 1)
                def _():
                    pltpu.sync_copy(h1_hbm.at[s16], hbuf)

                for k in range(Hh2):
                    kg = bcast(cid * Hh2 + k)
                    vs = plsc.load_gather(asbuf, [lane, kg])
                    vd = plsc.load_gather(adbuf, [lane, kg + 8])
                    ev = vs + vd
                    ev = jnp.where(ev >= 0.0, ev, 0.2 * ev)
                    w = jnp.where(valid, jnp.exp(ev), 0.0)
                    plsc.store_scatter(wbuf, [lane, kg], w)

                for e in range(16):
                    for k in range(Hh2):
                        kg = bcast(cid * Hh2 + k)
                        wsv = plsc.load_gather(
                            wbuf, [jnp.full((LANES,), e, i32), kg])

                        def cs(j, _, e=e, k=k, wsv=wsv):
                            off = k * C + j * LANES
                            hbuf[e, pl.ds(off, LANES)] = (
                                hbuf[e, pl.ds(off, LANES)] * wsv)
                            return 0
                        lax.fori_loop(0, CV, cs, 0, unroll=4)

                idxw[...] = dl16
                pltpu.sync_copy(hbuf, acc_sh.at[idxw], add=True)
                pltpu.sync_copy(wbuf, accw_sh.at[idxw], add=True)
                return 0
            lax.fori_loop(0, ngrp, proc, 0)

            plsc.subcore_barrier()

            rows_sh = pl.ds(sid * RPT, RPT)
            rows_g = pl.ds(lo + sid * RPT, RPT)

            @pl.when(cid == 0)
            def _():
                pltpu.sync_copy(acc_sh.at[rows_sh], acc0_hbm.at[rows_g])
                pltpu.sync_copy(accw_sh.at[rows_sh], accw0_hbm.at[rows_g])

            @pl.when(cid == 1)
            def _():
                pltpu.sync_copy(acc_sh.at[rows_sh], acc1_hbm.at[rows_g])
                pltpu.sync_copy(accw_sh.at[rows_sh], accw1_hbm.at[rows_g])
            plsc.subcore_barrier()
            return 0

        lax.fori_loop(0, NB, blk, 0)

    return body(pedge, att, h0, h1, zrows, zw)


TRUNK_OUT = [448, 384, 256]
SC_CFG = {448: (320, 32), 384: (320, 32), 256: (640, 16), 64: (5120, 2)}


def _layer(h, pedge, W, a_src, a_dst, Hh, C):
    D2 = Hh // 2 * C
    h0 = _mm(h, W[:, :D2])
    h1 = _mm(h, W[:, D2:])
    att = _att(h0, h1, a_src, a_dst, Hh, C)
    RB, NB = SC_CFG[C]
    zrows = jnp.zeros((RB, D2), f32)
    zw = jnp.zeros((RB, 16), f32)
    return _sc_edge(pedge, att, h0, h1, zrows, zw, Hh, C, RB, NB)


def kernel(x, edge_index, params):
    xp = jnp.zeros((NP, x.shape[1]), f32).at[:N].set(x)
    pedge = edge_index[0] * PK + edge_index[1]

    h = xp
    for i in range(3):
        p = params["trunk"][i]
        C = TRUNK_OUT[i]
        acc0, acc1, accw0, accw1 = _layer(
            h, pedge, p["W"], p["att_src"].reshape(-1),
            p["att_dst"].reshape(-1), H, C)
        h = _norm(acc0, acc1, accw0, accw1,
                  p["b"], p["ln_g"], p["ln_b"], H, C)

    hp = params["heads"]
    Wf = jnp.concatenate([q["W"] for q in hp], axis=1)
    asf = jnp.concatenate([q["att_src"].reshape(-1) for q in hp], axis=0)
    adf = jnp.concatenate([q["att_dst"].reshape(-1) for q in hp], axis=0)
    bf = jnp.concatenate([q["b"] for q in hp], axis=0)
    acc0, acc1, accw0, accw1 = _layer(h, pedge, Wf, asf, adf, 4, 64)
    out = _final(acc0, acc1, accw0, accw1, bf, 4, 64)
    return out[:N]


# speedup vs baseline: 4.4832x; 1.1742x over previous
import functools

import jax
import jax.numpy as jnp
from jax import lax
from jax.experimental import pallas as pl
from jax.experimental.pallas import tpu as pltpu
from jax.experimental.pallas import tpu_sc as plsc

N = 10000
NP = 10240
E = 160000
H = 8
NC = 2
NS = 16
LANES = 16
PK = 16384

f32 = jnp.float32
i32 = jnp.int32


def _mm(z, w):
    M, K = z.shape
    _, Nn = w.shape
    bm, bk = 256, 256
    bn = 256 if Nn % 256 == 0 else 128
    gm, gn, gk = M // bm, Nn // bn, K // bk

    def body(zb, wb, ob):
        @pl.when(pl.program_id(2) == 0)
        def _():
            ob[...] = jnp.zeros_like(ob)
        ob[...] += jnp.dot(zb[...], wb[...], preferred_element_type=f32)

    return pl.pallas_call(
        body,
        grid=(gm, gn, gk),
        in_specs=[pl.BlockSpec((bm, bk), lambda i, j, k: (i, k)),
                  pl.BlockSpec((bk, bn), lambda i, j, k: (k, j))],
        out_specs=pl.BlockSpec((bm, bn), lambda i, j, k: (i, j)),
        out_shape=jax.ShapeDtypeStruct((M, Nn), f32),
        compiler_params=pltpu.CompilerParams(
            dimension_semantics=("parallel", "parallel", "arbitrary")),
    )(z, w)


def _att(h0, h1, a_src, a_dst, Hh, C):
    M, D2 = h0.shape
    Hh2 = Hh // 2
    bm = 256

    def body(h0b, h1b, asb, adb, ob):
        cols = []
        for vec, base in ((asb, 0), (adb, 8)):
            del base
            for k in range(Hh):
                hb = h0b if k < Hh2 else h1b
                kl = k % Hh2
                hs = hb[:, kl * C:(kl + 1) * C]
                cols.append(jnp.sum(hs * vec[0, k * C:(k + 1) * C][None, :],
                                    axis=1, keepdims=True))
            for _ in range(Hh, 8):
                cols.append(jnp.zeros((bm, 1), f32))
        ob[...] = jnp.concatenate(cols, axis=1)

    return pl.pallas_call(
        body,
        grid=(M // bm,),
        in_specs=[pl.BlockSpec((bm, D2), lambda i: (i, 0)),
                  pl.BlockSpec((bm, D2), lambda i: (i, 0)),
                  pl.BlockSpec((1, Hh * C), lambda i: (0, 0)),
                  pl.BlockSpec((1, Hh * C), lambda i: (0, 0))],
        out_specs=pl.BlockSpec((bm, 16), lambda i: (i, 0)),
        out_shape=jax.ShapeDtypeStruct((M, 16), f32),
    )(h0, h1, a_src.reshape(1, -1), a_dst.reshape(1, -1))


def _norm(acc0, acc1, accw0, accw1, b, g, lb, Hh, C):
    M, D2 = acc0.shape
    D = 2 * D2
    Hh2 = Hh // 2
    bm = 256

    def body(a0, a1, w0, w1, bb, gb, lbb, ob):
        s = w0[...] + w1[...]
        parts = []
        for k in range(Hh):
            ab = a0 if k < Hh2 else a1
            kl = k % Hh2
            parts.append(ab[:, kl * C:(kl + 1) * C]
                         / (s[:, k:k + 1] + 1e-16))
        y = jnp.concatenate(parts, axis=1) + bb[0][None, :]
        y = jnp.maximum(y, 0.0)
        mu = jnp.mean(y, axis=1, keepdims=True)
        var = jnp.mean((y - mu) * (y - mu), axis=1, keepdims=True)
        ob[...] = (y - mu) * lax.rsqrt(var + 1e-5) * gb[0][None, :] + lbb[0][None, :]

    return pl.pallas_call(
        body,
        grid=(M // bm,),
        in_specs=[pl.BlockSpec((bm, D2), lambda i: (i, 0)),
                  pl.BlockSpec((bm, D2), lambda i: (i, 0)),
                  pl.BlockSpec((bm, 16), lambda i: (i, 0)),
                  pl.BlockSpec((bm, 16), lambda i: (i, 0)),
                  pl.BlockSpec((1, D), lambda i: (0, 0)),
                  pl.BlockSpec((1, D), lambda i: (0, 0)),
                  pl.BlockSpec((1, D), lambda i: (0, 0))],
        out_specs=pl.BlockSpec((bm, D), lambda i: (i, 0)),
        out_shape=jax.ShapeDtypeStruct((M, D), f32),
    )(acc0, acc1, accw0, accw1,
      b.reshape(1, -1), g.reshape(1, -1), lb.reshape(1, -1))


def _final(acc0, acc1, accw0, accw1, b, Hh, C):
    M, D2 = acc0.shape
    D = 2 * D2
    Hh2 = Hh // 2
    bm = 256

    def body(a0, a1, w0, w1, bb, ob):
        s = w0[...] + w1[...]
        parts = []
        for k in range(Hh):
            ab = a0 if k < Hh2 else a1
            kl = k % Hh2
            parts.append(ab[:, kl * C:(kl + 1) * C]
                         / (s[:, k:k + 1] + 1e-16))
        ob[...] = jnp.concatenate(parts, axis=1) + bb[0][None, :]

    return pl.pallas_call(
        body,
        grid=(M // bm,),
        in_specs=[pl.BlockSpec((bm, D2), lambda i: (i, 0)),
                  pl.BlockSpec((bm, D2), lambda i: (i, 0)),
                  pl.BlockSpec((bm, 16), lambda i: (i, 0)),
                  pl.BlockSpec((bm, 16), lambda i: (i, 0)),
                  pl.BlockSpec((1, D), lambda i: (0, 0))],
        out_specs=pl.BlockSpec((bm, D), lambda i: (i, 0)),
        out_shape=jax.ShapeDtypeStruct((M, D), f32),
    )(acc0, acc1, accw0, accw1, b.reshape(1, -1))


def _sc_edge(pedge, att, h0, h1, zrows, zw, Hh, C, RB, NB):
    D2 = Hh // 2 * C
    Hh2 = Hh // 2
    EPT = E // NS
    NGR = EPT // LANES
    CV = C // LANES
    RPT = RB // NS

    mesh = plsc.VectorSubcoreMesh(core_axis_name="c", subcore_axis_name="s",
                                  num_cores=NC, num_subcores=NS)

    @functools.partial(
        pl.kernel,
        out_type=[jax.ShapeDtypeStruct((NP, D2), f32),
                  jax.ShapeDtypeStruct((NP, D2), f32),
                  jax.ShapeDtypeStruct((NP, 16), f32),
                  jax.ShapeDtypeStruct((NP, 16), f32)],
        mesh=mesh,
        compiler_params=pltpu.CompilerParams(needs_layout_passes=False,
                                             use_tc_tiling_on_sc=False),
        scratch_types=[
            pltpu.VMEM((EPT,), i32),
            pltpu.VMEM((EPT + 16,), i32),
            pltpu.VMEM((16, 16), f32),
            pltpu.VMEM((16, 16), f32),
            pltpu.VMEM((16, 16), f32),
            pltpu.VMEM((2, 8, D2), f32),
            pltpu.VMEM((16,), i32),
            pltpu.VMEM((2, 8), i32),
            pltpu.VMEM((2, 8), i32),
            pltpu.VMEM_SHARED((RB, D2), f32),
            pltpu.VMEM_SHARED((RB, 16), f32),
            pltpu.SemaphoreType.DMA,
            pltpu.SemaphoreType.DMA,
            pltpu.SemaphoreType.DMA,
            pltpu.SemaphoreType.DMA,
            pltpu.SemaphoreType.DMA,
            pltpu.SemaphoreType.DMA,
            pltpu.SemaphoreType.DMA,
        ],
    )
    def body(pedge_hbm, att_hbm, h0_hbm, h1_hbm, z_hbm, zw_hbm,
             acc0_hbm, acc1_hbm, accw0_hbm, accw1_hbm,
             pedge_v, cpk, asbuf, adbuf, wbuf, hbuf, idxw, idxw2, ssrc2,
             acc_sh, accw_sh,
             sem_h0, sem_h1, sem_as, sem_ad, sem_s0, sem_s1, sem_sw):
        cid = lax.axis_index("c")
        sid = lax.axis_index("s")

        def bcast(v):
            return lax.broadcast(v, (LANES,))

        pltpu.sync_copy(pedge_hbm.at[pl.ds(sid * EPT, EPT)], pedge_v)
        for r in range(16):
            wbuf[r, :] = jnp.zeros((LANES,), f32)

        def blk(bi, _):
            lo = bi * RB
            hi = lo + RB

            pltpu.sync_copy(z_hbm.at[pl.ds(sid * RPT, RPT)],
                            acc_sh.at[pl.ds(sid * RPT, RPT)])
            pltpu.sync_copy(zw_hbm.at[pl.ds(sid * RPT, RPT)],
                            accw_sh.at[pl.ds(sid * RPT, RPT)])
            plsc.subcore_barrier()

            def scan(g, ptr):
                sl = pl.ds(g * LANES, LANES)
                p16 = pedge_v[sl]
                d16 = p16 & jnp.full((LANES,), PK - 1, i32)
                inb = (d16 >= bcast(lo)) & (d16 < bcast(hi))
                cs = plsc.cumsum(inb.astype(i32))
                pos = jnp.where(inb, bcast(ptr) + cs - 1,
                                jnp.full((LANES,), EPT + 8, i32))
                plsc.store_scatter(cpk, [pos], p16 - bcast(lo))
                cnt = plsc.all_reduce_population_count(inb)
                if cnt.ndim:
                    cnt = cnt[0]
                return ptr + cnt
            ecnt = lax.fori_loop(0, NGR, scan, jnp.int32(0))

            ngrp = (ecnt + LANES - 1) // LANES

            def wait_scatters():
                pltpu.make_async_copy(
                    hbuf.at[0], acc_sh.at[idxw2.at[0]], sem_s0).wait()
                pltpu.make_async_copy(
                    hbuf.at[1], acc_sh.at[idxw2.at[1]], sem_s1).wait()
                pltpu.make_async_copy(
                    wbuf, accw_sh.at[idxw], sem_sw).wait()

            def proc(g, _):
                lane = lax.iota(i32, LANES)
                sl = pl.ds(g * LANES, LANES)
                valid = (bcast(g * LANES) + lane) < bcast(ecnt)
                p16 = jnp.where(valid, cpk[sl], 0)
                s16 = jnp.clip(
                    lax.shift_right_logical(p16, jnp.full((LANES,), 14, i32)),
                    0, N - 1)
                dl16 = jnp.clip(p16 & jnp.full((LANES,), PK - 1, i32),
                                0, RB - 1)
                dg16 = jnp.clip(dl16 + bcast(lo), 0, NP - 1)

                @pl.when(g > 0)
                def _():
                    wait_scatters()

                row = lax.shift_right_logical(lane, jnp.full((LANES,), 3, i32))
                col = lane & jnp.full((LANES,), 7, i32)
                plsc.store_scatter(ssrc2, [row, col], s16)
                plsc.store_scatter(idxw2, [row, col], dl16)
                idxw[...] = dl16

                @pl.when(cid == 0)
                def _():
                    pltpu.async_copy(h0_hbm.at[ssrc2.at[0]], hbuf.at[0], sem_h0)
                    pltpu.async_copy(h0_hbm.at[ssrc2.at[1]], hbuf.at[1], sem_h1)

                @pl.when(cid == 1)
                def _():
                    pltpu.async_copy(h1_hbm.at[ssrc2.at[0]], hbuf.at[0], sem_h0)
                    pltpu.async_copy(h1_hbm.at[ssrc2.at[1]], hbuf.at[1], sem_h1)
                pltpu.async_copy(att_hbm.at[s16], asbuf, sem_as)
                pltpu.async_copy(att_hbm.at[dg16], adbuf, sem_ad)

                pltpu.make_async_copy(att_hbm.at[s16], asbuf, sem_as).wait()
                pltpu.make_async_copy(att_hbm.at[dg16], adbuf, sem_ad).wait()
                for k in range(Hh2):
                    kg = bcast(cid * Hh2 + k)
                    vs = plsc.load_gather(asbuf, [lane, kg])
                    vd = plsc.load_gather(adbuf, [lane, kg + 8])
                    ev = vs + vd
                    ev = jnp.where(ev >= 0.0, ev, 0.2 * ev)
                    w = jnp.where(valid, jnp.exp(ev), 0.0)
                    plsc.store_scatter(wbuf, [lane, kg], w)
                pltpu.async_copy(wbuf, accw_sh.at[idxw], sem_sw, add=True)

                for hf in range(2):
                    pltpu.make_async_copy(
                        h0_hbm.at[ssrc2.at[hf]], hbuf.at[hf],
                        sem_h0 if hf == 0 else sem_h1).wait()
                    for e in range(8):
                        for k in range(Hh2):
                            kg = bcast(cid * Hh2 + k)
                            wsv = plsc.load_gather(
                                wbuf,
                                [jnp.full((LANES,), hf * 8 + e, i32), kg])

                            def cs(j, _, hf=hf, e=e, k=k, wsv=wsv):
                                off = k * C + j * LANES
                                hbuf[hf, e, pl.ds(off, LANES)] = (
                                    hbuf[hf, e, pl.ds(off, LANES)] * wsv)
                                return 0
                            lax.fori_loop(0, CV, cs, 0, unroll=4)
                    pltpu.async_copy(
                        hbuf.at[hf], acc_sh.at[idxw2.at[hf]],
                        sem_s0 if hf == 0 else sem_s1, add=True)
                return 0
            lax.fori_loop(0, ngrp, proc, 0)

            @pl.when(ngrp > 0)
            def _():
                wait_scatters()

            plsc.subcore_barrier()

            rows_sh = pl.ds(sid * RPT, RPT)
            rows_g = pl.ds(lo + sid * RPT, RPT)

            @pl.when(cid == 0)
            def _():
                pltpu.sync_copy(acc_sh.at[rows_sh], acc0_hbm.at[rows_g])
                pltpu.sync_copy(accw_sh.at[rows_sh], accw0_hbm.at[rows_g])

            @pl.when(cid == 1)
            def _():
                pltpu.sync_copy(acc_sh.at[rows_sh], acc1_hbm.at[rows_g])
                pltpu.sync_copy(accw_sh.at[rows_sh], accw1_hbm.at[rows_g])
            plsc.subcore_barrier()
            return 0

        lax.fori_loop(0, NB, blk, 0)

    return body(pedge, att, h0, h1, zrows, zw)


TRUNK_OUT = [448, 384, 256]
SC_CFG = {448: (320, 32), 384: (320, 32), 256: (640, 16), 64: (5120, 2)}


def _layer(h, pedge, W, a_src, a_dst, Hh, C):
    D2 = Hh // 2 * C
    h0 = _mm(h, W[:, :D2])
    h1 = _mm(h, W[:, D2:])
    att = _att(h0, h1, a_src, a_dst, Hh, C)
    RB, NB = SC_CFG[C]
    zrows = jnp.zeros((RB, D2), f32)
    zw = jnp.zeros((RB, 16), f32)
    return _sc_edge(pedge, att, h0, h1, zrows, zw, Hh, C, RB, NB)


def kernel(x, edge_index, params):
    xp = jnp.zeros((NP, x.shape[1]), f32).at[:N].set(x)
    pedge = edge_index[0] * PK + edge_index[1]

    h = xp
    for i in range(3):
        p = params["trunk"][i]
        C = TRUNK_OUT[i]
        acc0, acc1, accw0, accw1 = _layer(
            h, pedge, p["W"], p["att_src"].reshape(-1),
            p["att_dst"].reshape(-1), H, C)
        h = _norm(acc0, acc1, accw0, accw1,
                  p["b"], p["ln_g"], p["ln_b"], H, C)

    hp = params["heads"]
    Wf = jnp.concatenate([q["W"] for q in hp], axis=1)
    asf = jnp.concatenate([q["att_src"].reshape(-1) for q in hp], axis=0)
    adf = jnp.concatenate([q["att_dst"].reshape(-1) for q in hp], axis=0)
    bf = jnp.concatenate([q["b"] for q in hp], axis=0)
    acc0, acc1, accw0, accw1 = _layer(h, pedge, Wf, asf, adf, 4, 64)
    out = _final(acc0, acc1, accw0, accw1, bf, 4, 64)
    return out[:N]


# speedup vs baseline: 6.8198x; 1.5212x over previous
import functools

import jax
import jax.numpy as jnp
from jax import lax
from jax.experimental import pallas as pl
from jax.experimental.pallas import tpu as pltpu
from jax.experimental.pallas import tpu_sc as plsc

N = 10000
NP = 10240
E = 160000
H = 8
NC = 2
NS = 16
LANES = 16
PK = 16384

f32 = jnp.float32
i32 = jnp.int32


def _mm(z, w):
    M, K = z.shape
    _, Nn = w.shape
    bm = 1024
    bn = 512 if Nn % 512 == 0 else Nn
    gm, gn = M // bm, Nn // bn

    def body(zb, wb, ob):
        ob[...] = jnp.dot(zb[...], wb[...], preferred_element_type=f32)

    return pl.pallas_call(
        body,
        grid=(gm, gn),
        in_specs=[pl.BlockSpec((bm, K), lambda i, j: (i, 0)),
                  pl.BlockSpec((K, bn), lambda i, j: (0, j))],
        out_specs=pl.BlockSpec((bm, bn), lambda i, j: (i, j)),
        out_shape=jax.ShapeDtypeStruct((M, Nn), f32),
        compiler_params=pltpu.CompilerParams(
            dimension_semantics=("parallel", "arbitrary")),
    )(z, w)


def _att(h0, h1, a_src, a_dst, Hh, C):
    M, D2 = h0.shape
    Hh2 = Hh // 2
    bm = 256

    def body(h0b, h1b, asb, adb, ob):
        cols = []
        for vec, base in ((asb, 0), (adb, 8)):
            del base
            for k in range(Hh):
                hb = h0b if k < Hh2 else h1b
                kl = k % Hh2
                hs = hb[:, kl * C:(kl + 1) * C]
                cols.append(jnp.sum(hs * vec[0, k * C:(k + 1) * C][None, :],
                                    axis=1, keepdims=True))
            for _ in range(Hh, 8):
                cols.append(jnp.zeros((bm, 1), f32))
        ob[...] = jnp.concatenate(cols, axis=1)

    return pl.pallas_call(
        body,
        grid=(M // bm,),
        in_specs=[pl.BlockSpec((bm, D2), lambda i: (i, 0)),
                  pl.BlockSpec((bm, D2), lambda i: (i, 0)),
                  pl.BlockSpec((1, Hh * C), lambda i: (0, 0)),
                  pl.BlockSpec((1, Hh * C), lambda i: (0, 0))],
        out_specs=pl.BlockSpec((bm, 16), lambda i: (i, 0)),
        out_shape=jax.ShapeDtypeStruct((M, 16), f32),
    )(h0, h1, a_src.reshape(1, -1), a_dst.reshape(1, -1))


def _norm(acc0, acc1, accw0, accw1, b, g, lb, Hh, C):
    M, D2 = acc0.shape
    D = 2 * D2
    Hh2 = Hh // 2
    bm = 256

    def body(a0, a1, w0, w1, bb, gb, lbb, ob):
        s = w0[...] + w1[...]
        parts = []
        for k in range(Hh):
            ab = a0 if k < Hh2 else a1
            kl = k % Hh2
            parts.append(ab[:, kl * C:(kl + 1) * C]
                         / (s[:, k:k + 1] + 1e-16))
        y = jnp.concatenate(parts, axis=1) + bb[0][None, :]
        y = jnp.maximum(y, 0.0)
        mu = jnp.mean(y, axis=1, keepdims=True)
        var = jnp.mean((y - mu) * (y - mu), axis=1, keepdims=True)
        ob[...] = (y - mu) * lax.rsqrt(var + 1e-5) * gb[0][None, :] + lbb[0][None, :]

    return pl.pallas_call(
        body,
        grid=(M // bm,),
        in_specs=[pl.BlockSpec((bm, D2), lambda i: (i, 0)),
                  pl.BlockSpec((bm, D2), lambda i: (i, 0)),
                  pl.BlockSpec((bm, 16), lambda i: (i, 0)),
                  pl.BlockSpec((bm, 16), lambda i: (i, 0)),
                  pl.BlockSpec((1, D), lambda i: (0, 0)),
                  pl.BlockSpec((1, D), lambda i: (0, 0)),
                  pl.BlockSpec((1, D), lambda i: (0, 0))],
        out_specs=pl.BlockSpec((bm, D), lambda i: (i, 0)),
        out_shape=jax.ShapeDtypeStruct((M, D), f32),
    )(acc0, acc1, accw0, accw1,
      b.reshape(1, -1), g.reshape(1, -1), lb.reshape(1, -1))


def _final(acc0, acc1, accw0, accw1, b, Hh, C):
    M, D2 = acc0.shape
    D = 2 * D2
    Hh2 = Hh // 2
    bm = 256

    def body(a0, a1, w0, w1, bb, ob):
        s = w0[...] + w1[...]
        parts = []
        for k in range(Hh):
            ab = a0 if k < Hh2 else a1
            kl = k % Hh2
            parts.append(ab[:, kl * C:(kl + 1) * C]
                         / (s[:, k:k + 1] + 1e-16))
        ob[...] = jnp.concatenate(parts, axis=1) + bb[0][None, :]

    return pl.pallas_call(
        body,
        grid=(M // bm,),
        in_specs=[pl.BlockSpec((bm, D2), lambda i: (i, 0)),
                  pl.BlockSpec((bm, D2), lambda i: (i, 0)),
                  pl.BlockSpec((bm, 16), lambda i: (i, 0)),
                  pl.BlockSpec((bm, 16), lambda i: (i, 0)),
                  pl.BlockSpec((1, D), lambda i: (0, 0))],
        out_specs=pl.BlockSpec((bm, D), lambda i: (i, 0)),
        out_shape=jax.ShapeDtypeStruct((M, D), f32),
    )(acc0, acc1, accw0, accw1, b.reshape(1, -1))


def _sc_edge(pedge, att, h0, h1, zrows, zw, Hh, C, RB, NB):
    D2 = Hh // 2 * C
    Hh2 = Hh // 2
    EPT = E // NS
    NGR = EPT // LANES
    CV = C // LANES
    RPT = RB // NS

    mesh = plsc.VectorSubcoreMesh(core_axis_name="c", subcore_axis_name="s",
                                  num_cores=NC, num_subcores=NS)

    @functools.partial(
        pl.kernel,
        out_type=[jax.ShapeDtypeStruct((NP, D2), f32),
                  jax.ShapeDtypeStruct((NP, D2), f32),
                  jax.ShapeDtypeStruct((NP, 16), f32),
                  jax.ShapeDtypeStruct((NP, 16), f32)],
        mesh=mesh,
        compiler_params=pltpu.CompilerParams(needs_layout_passes=False,
                                             use_tc_tiling_on_sc=False),
        scratch_types=[
            pltpu.VMEM((EPT,), i32),
            pltpu.VMEM((EPT + 16,), i32),
            pltpu.VMEM((16, 16), f32),
            pltpu.VMEM((16, 16), f32),
            pltpu.VMEM((16, 16), f32),
            pltpu.VMEM((2, 8, D2), f32),
            pltpu.VMEM((16,), i32),
            pltpu.VMEM((2, 8), i32),
            pltpu.VMEM((2, 8), i32),
            pltpu.VMEM_SHARED((RB, D2), f32),
            pltpu.VMEM_SHARED((RB, 16), f32),
            pltpu.SemaphoreType.DMA,
            pltpu.SemaphoreType.DMA,
            pltpu.SemaphoreType.DMA,
            pltpu.SemaphoreType.DMA,
            pltpu.SemaphoreType.DMA,
            pltpu.SemaphoreType.DMA,
            pltpu.SemaphoreType.DMA,
        ],
    )
    def body(pedge_hbm, att_hbm, h0_hbm, h1_hbm, z_hbm, zw_hbm,
             acc0_hbm, acc1_hbm, accw0_hbm, accw1_hbm,
             pedge_v, cpk, asbuf, adbuf, wbuf, hbuf, idxw, idxw2, ssrc2,
             acc_sh, accw_sh,
             sem_h0, sem_h1, sem_as, sem_ad, sem_s0, sem_s1, sem_sw):
        cid = lax.axis_index("c")
        sid = lax.axis_index("s")

        def bcast(v):
            return lax.broadcast(v, (LANES,))

        pltpu.sync_copy(pedge_hbm.at[pl.ds(sid * EPT, EPT)], pedge_v)
        for r in range(16):
            wbuf[r, :] = jnp.zeros((LANES,), f32)

        def blk(bi, _):
            lo = bi * RB
            hi = lo + RB

            pltpu.sync_copy(z_hbm.at[pl.ds(sid * RPT, RPT)],
                            acc_sh.at[pl.ds(sid * RPT, RPT)])
            pltpu.sync_copy(zw_hbm.at[pl.ds(sid * RPT, RPT)],
                            accw_sh.at[pl.ds(sid * RPT, RPT)])
            plsc.subcore_barrier()

            def scan(g, ptr):
                sl = pl.ds(g * LANES, LANES)
                p16 = pedge_v[sl]
                d16 = p16 & jnp.full((LANES,), PK - 1, i32)
                inb = (d16 >= bcast(lo)) & (d16 < bcast(hi))
                cs = plsc.cumsum(inb.astype(i32))
                pos = jnp.where(inb, bcast(ptr) + cs - 1,
                                jnp.full((LANES,), EPT + 8, i32))
                plsc.store_scatter(cpk, [pos], p16 - bcast(lo))
                cnt = plsc.all_reduce_population_count(inb)
                if cnt.ndim:
                    cnt = cnt[0]
                return ptr + cnt
            ecnt = lax.fori_loop(0, NGR, scan, jnp.int32(0))

            ngrp = (ecnt + LANES - 1) // LANES

            def wait_scatters():
                pltpu.make_async_copy(
                    hbuf.at[0], acc_sh.at[idxw2.at[0]], sem_s0).wait()
                pltpu.make_async_copy(
                    hbuf.at[1], acc_sh.at[idxw2.at[1]], sem_s1).wait()
                pltpu.make_async_copy(
                    wbuf, accw_sh.at[idxw], sem_sw).wait()

            def proc(g, _):
                lane = lax.iota(i32, LANES)
                sl = pl.ds(g * LANES, LANES)
                valid = (bcast(g * LANES) + lane) < bcast(ecnt)
                p16 = jnp.where(valid, cpk[sl], 0)
                s16 = jnp.clip(
                    lax.shift_right_logical(p16, jnp.full((LANES,), 14, i32)),
                    0, N - 1)
                dl16 = jnp.clip(p16 & jnp.full((LANES,), PK - 1, i32),
                                0, RB - 1)
                dg16 = jnp.clip(dl16 + bcast(lo), 0, NP - 1)

                @pl.when(g > 0)
                def _():
                    wait_scatters()

                row = lax.shift_right_logical(lane, jnp.full((LANES,), 3, i32))
                col = lane & jnp.full((LANES,), 7, i32)
                plsc.store_scatter(ssrc2, [row, col], s16)
                plsc.store_scatter(idxw2, [row, col], dl16)
                idxw[...] = dl16

                @pl.when(cid == 0)
                def _():
                    pltpu.async_copy(h0_hbm.at[ssrc2.at[0]], hbuf.at[0], sem_h0)
                    pltpu.async_copy(h0_hbm.at[ssrc2.at[1]], hbuf.at[1], sem_h1)

                @pl.when(cid == 1)
                def _():
                    pltpu.async_copy(h1_hbm.at[ssrc2.at[0]], hbuf.at[0], sem_h0)
                    pltpu.async_copy(h1_hbm.at[ssrc2.at[1]], hbuf.at[1], sem_h1)
                pltpu.async_copy(att_hbm.at[s16], asbuf, sem_as)
                pltpu.async_copy(att_hbm.at[dg16], adbuf, sem_ad)

                pltpu.make_async_copy(att_hbm.at[s16], asbuf, sem_as).wait()
                pltpu.make_async_copy(att_hbm.at[dg16], adbuf, sem_ad).wait()
                for k in range(Hh2):
                    kg = bcast(cid * Hh2 + k)
                    vs = plsc.load_gather(asbuf, [lane, kg])
                    vd = plsc.load_gather(adbuf, [lane, kg + 8])
                    ev = vs + vd
                    ev = jnp.where(ev >= 0.0, ev, 0.2 * ev)
                    w = jnp.where(valid, jnp.exp(ev), 0.0)
                    plsc.store_scatter(wbuf, [lane, kg], w)
                pltpu.async_copy(wbuf, accw_sh.at[idxw], sem_sw, add=True)

                for hf in range(2):
                    pltpu.make_async_copy(
                        h0_hbm.at[ssrc2.at[hf]], hbuf.at[hf],
                        sem_h0 if hf == 0 else sem_h1).wait()
                    for e in range(8):
                        for k in range(Hh2):
                            kg = bcast(cid * Hh2 + k)
                            wsv = plsc.load_gather(
                                wbuf,
                                [jnp.full((LANES,), hf * 8 + e, i32), kg])

                            def cs(j, _, hf=hf, e=e, k=k, wsv=wsv):
                                off = k * C + j * LANES
                                hbuf[hf, e, pl.ds(off, LANES)] = (
                                    hbuf[hf, e, pl.ds(off, LANES)] * wsv)
                                return 0
                            lax.fori_loop(0, CV, cs, 0, unroll=4)
                    pltpu.async_copy(
                        hbuf.at[hf], acc_sh.at[idxw2.at[hf]],
                        sem_s0 if hf == 0 else sem_s1, add=True)
                return 0
            lax.fori_loop(0, ngrp, proc, 0)

            @pl.when(ngrp > 0)
            def _():
                wait_scatters()

            plsc.subcore_barrier()

            rows_sh = pl.ds(sid * RPT, RPT)
            rows_g = pl.ds(lo + sid * RPT, RPT)

            @pl.when(cid == 0)
            def _():
                pltpu.sync_copy(acc_sh.at[rows_sh], acc0_hbm.at[rows_g])
                pltpu.sync_copy(accw_sh.at[rows_sh], accw0_hbm.at[rows_g])

            @pl.when(cid == 1)
            def _():
                pltpu.sync_copy(acc_sh.at[rows_sh], acc1_hbm.at[rows_g])
                pltpu.sync_copy(accw_sh.at[rows_sh], accw1_hbm.at[rows_g])
            plsc.subcore_barrier()
            return 0

        lax.fori_loop(0, NB, blk, 0)

    return body(pedge, att, h0, h1, zrows, zw)


TRUNK_OUT = [448, 384, 256]
SC_CFG = {448: (320, 32), 384: (320, 32), 256: (640, 16), 64: (5120, 2)}


def _layer(h, pedge, W, a_src, a_dst, Hh, C):
    D2 = Hh // 2 * C
    h0 = _mm(h, W[:, :D2])
    h1 = _mm(h, W[:, D2:])
    att = _att(h0, h1, a_src, a_dst, Hh, C)
    RB, NB = SC_CFG[C]
    zrows = jnp.zeros((RB, D2), f32)
    zw = jnp.zeros((RB, 16), f32)
    return _sc_edge(pedge, att, h0, h1, zrows, zw, Hh, C, RB, NB)


def kernel(x, edge_index, params):
    xp = jnp.zeros((NP, x.shape[1]), f32).at[:N].set(x)
    pedge = edge_index[0] * PK + edge_index[1]

    h = xp
    for i in range(3):
        p = params["trunk"][i]
        C = TRUNK_OUT[i]
        acc0, acc1, accw0, accw1 = _layer(
            h, pedge, p["W"], p["att_src"].reshape(-1),
            p["att_dst"].reshape(-1), H, C)
        h = _norm(acc0, acc1, accw0, accw1,
                  p["b"], p["ln_g"], p["ln_b"], H, C)

    hp = params["heads"]
    Wf = jnp.concatenate([q["W"] for q in hp], axis=1)
    asf = jnp.concatenate([q["att_src"].reshape(-1) for q in hp], axis=0)
    adf = jnp.concatenate([q["att_dst"].reshape(-1) for q in hp], axis=0)
    bf = jnp.concatenate([q["b"] for q in hp], axis=0)
    acc0, acc1, accw0, accw1 = _layer(h, pedge, Wf, asf, adf, 4, 64)
    out = _final(acc0, acc1, accw0, accw1, bf, 4, 64)
    return out[:N]
